# baseline (speedup 1.0000x reference)
import jax
import jax.numpy as jnp
from jax.experimental import pallas as pl
from jax.experimental.pallas import tpu as pltpu


def _body(lbl_ref, x_ref, ctx_ref, w1_ref, b1_ref, w2_ref, b2_ref,
          pre_ref, suf_ref, out_ref):
    h = jnp.maximum(
        jnp.dot(x_ref[0], w1_ref[...], preferred_element_type=jnp.float32)
        + b1_ref[...], 0.0)
    bias = jnp.dot(h, w2_ref[...], preferred_element_type=jnp.float32) + b2_ref[...]
    n_pre = pre_ref.shape[1]
    n_ctx = ctx_ref.shape[1]
    out_ref[0, 0:n_pre, :] = pre_ref[0]
    out_ref[0, n_pre:n_pre + n_ctx, :] = ctx_ref[0] + bias
    out_ref[0, n_pre + n_ctx:, :] = suf_ref[0]


def kernel(label, image_features, ctx, W1, b1, W2, b2, token_prefix, token_suffix):
    B = label.shape[0]
    num_classes, n_ctx, ctx_dim = ctx.shape
    vis_dim = image_features.shape[1]
    hid = W1.shape[1]
    pre_len = token_prefix.shape[1]
    suf_len = token_suffix.shape[1]
    seq = pre_len + n_ctx + suf_len

    b1_2d = b1.reshape(1, hid)
    b2_2d = b2.reshape(1, ctx_dim)
    x3d = image_features.reshape(B, 1, vis_dim)

    grid_spec = pltpu.PrefetchScalarGridSpec(
        num_scalar_prefetch=1,
        grid=(B,),
        in_specs=[
            pl.BlockSpec((1, 1, vis_dim), lambda i, lbl: (i, 0, 0)),
            pl.BlockSpec((1, n_ctx, ctx_dim), lambda i, lbl: (lbl[i], 0, 0)),
            pl.BlockSpec((vis_dim, hid), lambda i, lbl: (0, 0)),
            pl.BlockSpec((1, hid), lambda i, lbl: (0, 0)),
            pl.BlockSpec((hid, ctx_dim), lambda i, lbl: (0, 0)),
            pl.BlockSpec((1, ctx_dim), lambda i, lbl: (0, 0)),
            pl.BlockSpec((1, pre_len, ctx_dim), lambda i, lbl: (0, 0, 0)),
            pl.BlockSpec((1, suf_len, ctx_dim), lambda i, lbl: (0, 0, 0)),
        ],
        out_specs=pl.BlockSpec((1, seq, ctx_dim), lambda i, lbl: (i, 0, 0)),
    )

    out = pl.pallas_call(
        _body,
        grid_spec=grid_spec,
        out_shape=jax.ShapeDtypeStruct((B, seq, ctx_dim), jnp.float32),
    )(label.astype(jnp.int32), x3d, ctx, W1, b1_2d, W2, b2_2d,
      token_prefix, token_suffix)
    return out


# speedup vs baseline: 3.7814x; 3.7814x over previous
import jax
import jax.numpy as jnp
from jax.experimental import pallas as pl
from jax.experimental.pallas import tpu as pltpu

_BB = 16


def _body(lbl_ref, x_ref, w1_ref, b1_ref, w2_ref, b2_ref, pre_ref, suf_ref,
          ctx_any, out_ref, gbuf, gsem):
    nb = pl.num_programs(0)
    i = pl.program_id(0)
    slot = jax.lax.rem(i, 2)

    def start(s, step):
        for j in range(_BB):
            pltpu.make_async_copy(
                ctx_any.at[lbl_ref[step * _BB + j]],
                gbuf.at[s, j],
                gsem.at[s, j],
            ).start()

    @pl.when(i == 0)
    def _():
        start(0, 0)

    @pl.when(i + 1 < nb)
    def _():
        start(1 - slot, i + 1)

    for j in range(_BB):
        pltpu.make_async_copy(ctx_any.at[0], gbuf.at[slot, j],
                              gsem.at[slot, j]).wait()

    h = jnp.maximum(
        jnp.dot(x_ref[...], w1_ref[...], preferred_element_type=jnp.float32)
        + b1_ref[...], 0.0)
    bias = jnp.dot(h, w2_ref[...], preferred_element_type=jnp.float32) + b2_ref[...]

    n_pre = pre_ref.shape[0]
    n_ctx = gbuf.shape[2]
    ctx_sel = gbuf[slot] + bias[:, None, :]
    out_ref[:, 0:n_pre, :] = jnp.broadcast_to(pre_ref[None], (_BB,) + pre_ref.shape)
    out_ref[:, n_pre:n_pre + n_ctx, :] = ctx_sel
    out_ref[:, n_pre + n_ctx:, :] = jnp.broadcast_to(suf_ref[None], (_BB,) + suf_ref.shape)


def kernel(label, image_features, ctx, W1, b1, W2, b2, token_prefix, token_suffix):
    B = label.shape[0]
    num_classes, n_ctx, ctx_dim = ctx.shape
    vis_dim = image_features.shape[1]
    hid = W1.shape[1]
    pre_len = token_prefix.shape[1]
    suf_len = token_suffix.shape[1]
    seq = pre_len + n_ctx + suf_len
    nb = B // _BB

    grid_spec = pltpu.PrefetchScalarGridSpec(
        num_scalar_prefetch=1,
        grid=(nb,),
        in_specs=[
            pl.BlockSpec((_BB, vis_dim), lambda i, lbl: (i, 0)),
            pl.BlockSpec((vis_dim, hid), lambda i, lbl: (0, 0)),
            pl.BlockSpec((1, hid), lambda i, lbl: (0, 0)),
            pl.BlockSpec((hid, ctx_dim), lambda i, lbl: (0, 0)),
            pl.BlockSpec((1, ctx_dim), lambda i, lbl: (0, 0)),
            pl.BlockSpec((pre_len, ctx_dim), lambda i, lbl: (0, 0)),
            pl.BlockSpec((suf_len, ctx_dim), lambda i, lbl: (0, 0)),
            pl.BlockSpec(memory_space=pl.ANY),
        ],
        out_specs=pl.BlockSpec((_BB, seq, ctx_dim), lambda i, lbl: (i, 0, 0)),
        scratch_shapes=[
            pltpu.VMEM((2, _BB, n_ctx, ctx_dim), jnp.float32),
            pltpu.SemaphoreType.DMA((2, _BB)),
        ],
    )

    out = pl.pallas_call(
        _body,
        grid_spec=grid_spec,
        out_shape=jax.ShapeDtypeStruct((B, seq, ctx_dim), jnp.float32),
    )(label.astype(jnp.int32), image_features, W1, b1.reshape(1, hid), W2,
      b2.reshape(1, ctx_dim), token_prefix.reshape(pre_len, ctx_dim),
      token_suffix.reshape(suf_len, ctx_dim), ctx)
    return out
